# baseline (speedup 1.0000x reference)
import jax
import jax.numpy as jnp
from jax.experimental import pallas as pl

N = 256
OUT_W = N * (N + 1) // 2
BATCH_BLK = 64
H = N // 2


def _seg_off(i):
    return i * N - i * (i - 1) // 2


def _body(xa_ref, xb_ref, o_ref):
    for tr in range(N // 8):
        if tr < H // 8:
            blk = jnp.swapaxes(xa_ref[:, 8 * tr : 8 * tr + 8, :], 0, 1)
        else:
            blk = jnp.swapaxes(
                xb_ref[:, 8 * tr - H : 8 * tr - H + 8, :], 0, 1
            )
        for s in range(8):
            i = 8 * tr + s
            m = N - i
            col0 = i if i < H else i - H
            o_ref[:, pl.ds(_seg_off(i), m)] = blk[s, :, col0:]


def kernel(input):
    B = input.shape[0]
    out = pl.pallas_call(
        _body,
        grid=(B // BATCH_BLK,),
        in_specs=[
            pl.BlockSpec((BATCH_BLK, H, N), lambda b: (b, 0, 0)),
            pl.BlockSpec((BATCH_BLK, H, H), lambda b: (b, 1, 1)),
        ],
        out_specs=pl.BlockSpec((BATCH_BLK, OUT_W), lambda b: (b, 0)),
        out_shape=jax.ShapeDtypeStruct((B, OUT_W), input.dtype),
    )(input, input)
    return out
